# baseline (speedup 1.0000x reference)
import functools

import jax
import jax.numpy as jnp
import numpy as np
from jax import lax
from jax.experimental import pallas as pl
from jax.experimental.pallas import tpu as pltpu
from jax.experimental.pallas import tpu_sc as plsc

_B = 64
_D = 192
_SCALE_NS = (16, 64, 256)
_K = 4
_C = 16
_L = 16
_NW = 32


@functools.lru_cache(maxsize=1)
def _rng_consts():
    with jax.ensure_compile_time_eval():
        base = jax.random.key(42)
        idx0_cols = []
        expgs = []
        for si, n in enumerate(_SCALE_NS):
            keys = jax.random.split(jax.random.fold_in(base, si), _B)

            def per_key(kk, n=n):
                key, sub = jax.random.split(kk)
                i0 = jax.random.randint(sub, (), 0, n)
                gs = []
                for _ in range(_K - 1):
                    key, sub = jax.random.split(key)
                    gs.append(jax.random.gumbel(sub, (n,), jnp.float32))
                return i0, jnp.exp(jnp.stack(gs, axis=0))

            i0, g = jax.vmap(per_key)(keys)
            idx0_cols.append(np.asarray(i0, np.int32))
            expgs.append(np.asarray(g, np.float32))
        idx0s = np.stack(idx0_cols, axis=1).reshape(-1)
        idx0s = np.concatenate([idx0s, np.zeros((_L,), np.int32)])
        return idx0s, expgs


def _pool_matrix(g_out, g_in):
    m = np.zeros((g_out * g_out, g_in * g_in), np.float32)
    for n in range(g_in * g_in):
        a, bcol = n // g_in, n % g_in
        m[(a // 2) * g_out + (bcol // 2), n] = 0.25
    return m


def _pool_body(x_ref, a8_ref, a4_ref, o4_ref, o8_ref, o16_ref):
    x = x_ref[...]
    sh = x[:, :, 0] + x[:, :, 1]
    p16 = (sh[:, :, :, :_D] + sh[:, :, :, _D:]) * 0.25
    for i in range(16):
        o16_ref[:, pl.ds(i * 16, 16), :] = p16[:, i]
    for j in range(_C):
        p8j = jnp.dot(a8_ref[...], o16_ref[j], preferred_element_type=jnp.float32)
        o8_ref[j, :, :] = p8j
        o4_ref[j, :, :] = jnp.dot(a4_ref[...], p8j,
                                  preferred_element_type=jnp.float32)


def _sc_task(n, f_v, eg_v, idx0, rows_v, row_base):
    npc = n // _L
    iota = lax.broadcasted_iota(jnp.int32, (_L,), 0)
    inf_v = jnp.full((_L,), jnp.float32(np.inf), jnp.float32)
    minb = [inf_v] * npc
    idx = idx0
    for t in range(_K):
        for k in range(_D // _L):
            rows_v[row_base + t, pl.ds(k * _L, _L)] = f_v[idx, pl.ds(k * _L, _L)]
        if t == _K - 1:
            break
        idx_splat = jnp.full((_L,), 0, jnp.int32) + idx

        def dbody(d, accs, idx_splat=idx_splat):
            d_splat = jnp.full((_L,), 0, jnp.int32) + d
            cdv = plsc.load_gather(f_v, [idx_splat, d_splat])
            out = []
            for pc in range(npc):
                v = plsc.load_gather(f_v, [iota + (pc * _L), d_splat])
                dv = v - cdv
                out.append(accs[pc] + dv * dv)
            return tuple(out)

        accs = lax.fori_loop(
            0, _D, dbody,
            tuple(jnp.zeros((_L,), jnp.float32) for _ in range(npc)),
            unroll=4)
        ws = [None] * npc
        mv = jnp.full((_L,), jnp.float32(-np.inf), jnp.float32)
        for pc in range(npc):
            minb[pc] = jnp.minimum(minb[pc], accs[pc])
            w = (minb[pc] + 1e-38) * eg_v[t, pl.ds(pc * _L, _L)]
            ws[pc] = w
            mv = jnp.maximum(mv, w)
        mm = jnp.max(mv)
        idx_new = jnp.int32(n)
        for pc in range(npc - 1, -1, -1):
            cand = jnp.where(ws[pc] >= mm, iota + (pc * _L),
                             jnp.full((_L,), n, jnp.int32))
            cmin = jnp.min(cand)
            idx_new = jnp.where(cmin < n, cmin, idx_new)
        idx = idx_new
    return


def _sc_kernel(p4, p8, p16, eg4c, eg8c, eg16c, idx0c):
    mesh = plsc.VectorSubcoreMesh(core_axis_name="c", subcore_axis_name="s")

    @functools.partial(
        pl.kernel, mesh=mesh,
        compiler_params=pltpu.CompilerParams(needs_layout_passes=False),
        out_type=jax.ShapeDtypeStruct((_B, 3 * _K, _D), jnp.float32),
        scratch_types=[
            pltpu.VMEM((_SCALE_NS[0], _D), jnp.float32),
            pltpu.VMEM((_SCALE_NS[1], _D), jnp.float32),
            pltpu.VMEM((_SCALE_NS[2], _D), jnp.float32),
            pltpu.VMEM((_K - 1, _SCALE_NS[0]), jnp.float32),
            pltpu.VMEM((_K - 1, _SCALE_NS[1]), jnp.float32),
            pltpu.VMEM((_K - 1, _SCALE_NS[2]), jnp.float32),
            pltpu.VMEM((3 * _K, _D), jnp.float32),
            pltpu.VMEM((_B * 3 + _L, ), jnp.int32),
        ],
    )
    def k(p4h, p8h, p16h, eg4h, eg8h, eg16h, idx0h, outh,
          f4_v, f8_v, f16_v, eg4_v, eg8_v, eg16_v, rows_v, idx_v):
        w = lax.axis_index("s") * 2 + lax.axis_index("c")
        pltpu.sync_copy(idx0h, idx_v)
        f_refs = (f4_v, f8_v, f16_v)
        eg_refs = (eg4_v, eg8_v, eg16_v)
        fh_refs = (p4h, p8h, p16h)
        egh_refs = (eg4h, eg8h, eg16h)

        def img_body(bi, carry):
            b = w * 2 + bi
            for si in range(3):
                pltpu.sync_copy(fh_refs[si].at[b], f_refs[si])
                pltpu.sync_copy(egh_refs[si].at[b], eg_refs[si])
                idx0 = idx_v[pl.ds(b * 3 + si, _L)][0]
                _sc_task(_SCALE_NS[si], f_refs[si], eg_refs[si], idx0,
                         rows_v, si * _K)
            pltpu.sync_copy(rows_v, outh.at[b])
            return carry

        lax.fori_loop(0, _B // _NW, img_body, jnp.int32(0))

    return k(p4, p8, p16, eg4c, eg8c, eg16c, idx0c)


def kernel(features):
    b, h, w, d = features.shape
    idx0s, expgs = _rng_consts()
    x = features.reshape(b, h // 2, 2, w // 2, 2 * d)

    n4, n8, n16 = _SCALE_NS
    grid_spec = pltpu.PrefetchScalarGridSpec(
        num_scalar_prefetch=0,
        grid=(b // _C,),
        in_specs=[
            pl.BlockSpec((_C, h // 2, 2, w // 2, 2 * d),
                         lambda i: (i, 0, 0, 0, 0)),
            pl.BlockSpec((64, 256), lambda i: (0, 0)),
            pl.BlockSpec((16, 64), lambda i: (0, 0)),
        ],
        out_specs=[
            pl.BlockSpec((_C, n4, d), lambda i: (i, 0, 0)),
            pl.BlockSpec((_C, n8, d), lambda i: (i, 0, 0)),
            pl.BlockSpec((_C, n16, d), lambda i: (i, 0, 0)),
        ],
    )
    p4, p8, p16 = pl.pallas_call(
        _pool_body,
        grid_spec=grid_spec,
        out_shape=[
            jax.ShapeDtypeStruct((b, n4, d), jnp.float32),
            jax.ShapeDtypeStruct((b, n8, d), jnp.float32),
            jax.ShapeDtypeStruct((b, n16, d), jnp.float32),
        ],
    )(x, jnp.asarray(_pool_matrix(8, 16)), jnp.asarray(_pool_matrix(4, 8)))

    return _sc_kernel(p4, p8, p16,
                      jnp.asarray(expgs[0]), jnp.asarray(expgs[1]),
                      jnp.asarray(expgs[2]), jnp.asarray(idx0s))


# speedup vs baseline: 2.0897x; 2.0897x over previous
import functools

import jax
import jax.numpy as jnp
import numpy as np
from jax import lax
from jax.experimental import pallas as pl
from jax.experimental.pallas import tpu as pltpu
from jax.experimental.pallas import tpu_sc as plsc

_B = 64
_D = 192
_SCALE_NS = (16, 64, 256)
_K = 4
_C = 16
_L = 16
_NW = 32


@functools.lru_cache(maxsize=1)
def _rng_consts():
    with jax.ensure_compile_time_eval():
        base = jax.random.key(42)
        idx0_cols = []
        expgs = []
        for si, n in enumerate(_SCALE_NS):
            keys = jax.random.split(jax.random.fold_in(base, si), _B)

            def per_key(kk, n=n):
                key, sub = jax.random.split(kk)
                i0 = jax.random.randint(sub, (), 0, n)
                gs = []
                for _ in range(_K - 1):
                    key, sub = jax.random.split(key)
                    gs.append(jax.random.gumbel(sub, (n,), jnp.float32))
                return i0, jnp.exp(jnp.stack(gs, axis=0))

            i0, g = jax.vmap(per_key)(keys)
            idx0_cols.append(np.asarray(i0, np.int32))
            expgs.append(np.asarray(g, np.float32))
        idx0s = np.stack(idx0_cols, axis=1).reshape(-1)
        idx0s = np.concatenate([idx0s, np.zeros((_L,), np.int32)])
        return idx0s, expgs


def _pool_matrix(g_out, g_in):
    m = np.zeros((g_out * g_out, g_in * g_in), np.float32)
    for n in range(g_in * g_in):
        a, bcol = n // g_in, n % g_in
        m[(a // 2) * g_out + (bcol // 2), n] = 0.25
    return m


def _pool_body(x_ref, a8_ref, a4_ref, e16_ref, o4_ref, o8_ref, o16_ref,
               pf_ref):
    x = x_ref[...]
    sh = x[:, :, 0] + x[:, :, 1]
    p16 = (sh[:, :, :, :_D] + sh[:, :, :, _D:]) * 0.25
    for i in range(16):
        pf_ref[:, pl.ds(i * 16, 16), :] = p16[:, i]
    dn_t0 = (((0,), (0,)), ((), ()))
    n4, n8, n16 = _SCALE_NS
    eye = e16_ref[...]
    for j in range(_C):
        p16j = pf_ref[j]
        o16_ref[j, :, :] = jax.lax.dot_general(
            p16j, eye, dn_t0, preferred_element_type=jnp.float32)
        p8j = jnp.dot(a8_ref[...], p16j, preferred_element_type=jnp.float32)
        o8_ref[j, :, :] = jax.lax.dot_general(
            p8j, eye[:n8, :n8], dn_t0, preferred_element_type=jnp.float32)
        p4j = jnp.dot(a4_ref[...], p8j, preferred_element_type=jnp.float32)
        o4_ref[j, :, :] = jax.lax.dot_general(
            p4j, eye[:n4, :n4], dn_t0, preferred_element_type=jnp.float32)


def _sc_task(n, f_v, eg_v, idx0, rows_v, row_base):
    npc = n // _L
    iota = lax.broadcasted_iota(jnp.int32, (_L,), 0)
    inf_v = jnp.full((_L,), jnp.float32(np.inf), jnp.float32)
    minb = [inf_v] * npc
    idx = idx0
    for t in range(_K):
        idx_splat = jnp.full((_L,), 0, jnp.int32) + idx
        for k in range(_D // _L):
            rows_v[row_base + t, pl.ds(k * _L, _L)] = plsc.load_gather(
                f_v, [iota + (k * _L), idx_splat])
        if t == _K - 1:
            break

        def dbody(d, accs, idx_splat=idx_splat):
            d_splat = jnp.full((_L,), 0, jnp.int32) + d
            cdv = plsc.load_gather(f_v, [d_splat, idx_splat])
            out = []
            for pc in range(npc):
                v = f_v[d, pl.ds(pc * _L, _L)]
                dv = v - cdv
                out.append(accs[pc] + dv * dv)
            return tuple(out)

        accs = lax.fori_loop(
            0, _D, dbody,
            tuple(jnp.zeros((_L,), jnp.float32) for _ in range(npc)),
            unroll=4)
        ws = [None] * npc
        mv = jnp.full((_L,), jnp.float32(-np.inf), jnp.float32)
        for pc in range(npc):
            minb[pc] = jnp.minimum(minb[pc], accs[pc])
            w = (minb[pc] + 1e-38) * eg_v[t, pl.ds(pc * _L, _L)]
            ws[pc] = w
            mv = jnp.maximum(mv, w)
        mm = jnp.max(mv)
        idx_new = jnp.int32(n)
        for pc in range(npc - 1, -1, -1):
            cand = jnp.where(ws[pc] >= mm, iota + (pc * _L),
                             jnp.full((_L,), n, jnp.int32))
            cmin = jnp.min(cand)
            idx_new = jnp.where(cmin < n, cmin, idx_new)
        idx = idx_new
    return


def _sc_kernel(p4, p8, p16, eg4c, eg8c, eg16c, idx0c):
    mesh = plsc.VectorSubcoreMesh(core_axis_name="c", subcore_axis_name="s")

    @functools.partial(
        pl.kernel, mesh=mesh,
        compiler_params=pltpu.CompilerParams(needs_layout_passes=False),
        out_type=jax.ShapeDtypeStruct((_B, 3 * _K, _D), jnp.float32),
        scratch_types=[
            pltpu.VMEM((_D, _SCALE_NS[0]), jnp.float32),
            pltpu.VMEM((_D, _SCALE_NS[1]), jnp.float32),
            pltpu.VMEM((_D, _SCALE_NS[2]), jnp.float32),
            pltpu.VMEM((_K - 1, _SCALE_NS[0]), jnp.float32),
            pltpu.VMEM((_K - 1, _SCALE_NS[1]), jnp.float32),
            pltpu.VMEM((_K - 1, _SCALE_NS[2]), jnp.float32),
            pltpu.VMEM((3 * _K, _D), jnp.float32),
            pltpu.VMEM((_B * 3 + _L, ), jnp.int32),
        ],
    )
    def k(p4h, p8h, p16h, eg4h, eg8h, eg16h, idx0h, outh,
          f4_v, f8_v, f16_v, eg4_v, eg8_v, eg16_v, rows_v, idx_v):
        w = lax.axis_index("s") * 2 + lax.axis_index("c")
        pltpu.sync_copy(idx0h, idx_v)
        f_refs = (f4_v, f8_v, f16_v)
        eg_refs = (eg4_v, eg8_v, eg16_v)
        fh_refs = (p4h, p8h, p16h)
        egh_refs = (eg4h, eg8h, eg16h)

        def img_body(bi, carry):
            b = w * 2 + bi
            for si in range(3):
                pltpu.sync_copy(fh_refs[si].at[b], f_refs[si])
                pltpu.sync_copy(egh_refs[si].at[b], eg_refs[si])
                idx0 = idx_v[pl.ds(b * 3 + si, _L)][0]
                _sc_task(_SCALE_NS[si], f_refs[si], eg_refs[si], idx0,
                         rows_v, si * _K)
            pltpu.sync_copy(rows_v, outh.at[b])
            return carry

        lax.fori_loop(0, _B // _NW, img_body, jnp.int32(0))

    return k(p4, p8, p16, eg4c, eg8c, eg16c, idx0c)


def kernel(features):
    b, h, w, d = features.shape
    idx0s, expgs = _rng_consts()
    x = features.reshape(b, h // 2, 2, w // 2, 2 * d)

    n4, n8, n16 = _SCALE_NS
    grid_spec = pltpu.PrefetchScalarGridSpec(
        num_scalar_prefetch=0,
        grid=(b // _C,),
        in_specs=[
            pl.BlockSpec((_C, h // 2, 2, w // 2, 2 * d),
                         lambda i: (i, 0, 0, 0, 0)),
            pl.BlockSpec((64, 256), lambda i: (0, 0)),
            pl.BlockSpec((16, 64), lambda i: (0, 0)),
            pl.BlockSpec((256, 256), lambda i: (0, 0)),
        ],
        out_specs=[
            pl.BlockSpec((_C, d, n4), lambda i: (i, 0, 0)),
            pl.BlockSpec((_C, d, n8), lambda i: (i, 0, 0)),
            pl.BlockSpec((_C, d, n16), lambda i: (i, 0, 0)),
        ],
        scratch_shapes=[pltpu.VMEM((_C, n16, d), jnp.float32)],
    )
    p4, p8, p16 = pl.pallas_call(
        _pool_body,
        grid_spec=grid_spec,
        out_shape=[
            jax.ShapeDtypeStruct((b, d, n4), jnp.float32),
            jax.ShapeDtypeStruct((b, d, n8), jnp.float32),
            jax.ShapeDtypeStruct((b, d, n16), jnp.float32),
        ],
    )(x, jnp.asarray(_pool_matrix(8, 16)), jnp.asarray(_pool_matrix(4, 8)),
      jnp.asarray(np.eye(256, dtype=np.float32)))

    return _sc_kernel(p4, p8, p16,
                      jnp.asarray(expgs[0]), jnp.asarray(expgs[1]),
                      jnp.asarray(expgs[2]), jnp.asarray(idx0s))
